# baseline (speedup 1.0000x reference)
import jax
import jax.numpy as jnp
from jax import lax
from jax.experimental import pallas as pl
from jax.experimental.pallas import tpu as pltpu
from jax.experimental.pallas import tpu_sc as plsc

_NC = 2
_NS = 16
_NW = _NC * _NS

_V = 1000000
_D = 64
_B = 4096
_L = 200

_VBLK = _V // 128 + 1
_LAST = _VBLK - 1


def _iota16():
    return lax.iota(jnp.int32, 16)


def _a_body(tab_t, pairs, in0, in1, tin, t20, t21, rs0, rs1, ws0, ws1):
    wid = lax.axis_index("s") * _NC + lax.axis_index("c")
    ins = (in0, in1)
    t2s = (t20, t21)
    rsems = (rs0, rs1)
    wsems = (ws0, ws1)

    rows = [dd * 16 + _iota16() for dd in range(4)]

    nit = (_VBLK + _NW - 1) // _NW

    def read_full(i, b):
        vb = wid + i * _NW
        v0 = pl.multiple_of(jnp.minimum(vb, _LAST - 1) * 128, 128)
        return pltpu.make_async_copy(
            tab_t.at[:, pl.ds(v0, 128)], ins[b], rsems[b]
        )

    def read_tail(b):
        return pltpu.make_async_copy(
            tab_t.at[:, pl.ds(_LAST * 128, _V - _LAST * 128)], tin, rsems[b]
        )

    def read_start(i, b):
        vb = wid + i * _NW

        @pl.when(vb < _LAST)
        def _():
            read_full(i, b).start()

        @pl.when(vb == _LAST)
        def _():
            read_tail(b).start()

    def read_wait(i, b):
        vb = wid + i * _NW

        @pl.when(vb < _LAST)
        def _():
            read_full(i, b).wait()

        @pl.when(vb == _LAST)
        def _():
            read_tail(b).wait()

    def write_full(i, b):
        vb = wid + i * _NW
        return pltpu.make_async_copy(
            t2s[b], pairs.at[pl.ds(vb * 64, 64)], wsems[b]
        )

    def write_tail(b):
        return pltpu.make_async_copy(
            t2s[b].at[pl.ds(0, 32)],
            pairs.at[pl.ds((_V - 64) // 2, 32)],
            wsems[b],
        )

    def write_start(i, b):
        vb = wid + i * _NW

        @pl.when(vb < _LAST)
        def _():
            write_full(i, b).start()

        @pl.when(vb == _LAST)
        def _():
            write_tail(b).start()

    def write_wait(i, b):
        vb = wid + i * _NW

        @pl.when(vb < _LAST)
        def _():
            write_full(i, b).wait()

        @pl.when(vb == _LAST)
        def _():
            write_tail(b).wait()

    read_start(0, 0)

    def step(i, b):
        vb = wid + i * _NW

        @pl.when(vb <= _LAST)
        def _():
            @pl.when(vb + _NW <= _LAST)
            def _():
                read_start(i + 1, (b + 1) % 2)

            @pl.when(i >= 2)
            def _():
                write_wait(i - 2, b)

            read_wait(i, b)
            t2 = t2s[b]

            def col(src, v, prow, carry):
                h = lax.mul(lax.bitwise_and(v, 1), 64)
                for dd in range(4):
                    vals = plsc.load_gather(
                        src, [rows[dd], jnp.full((16,), v, jnp.int32)]
                    )
                    t2[prow, pl.ds(h + dd * 16, 16)] = vals
                return carry

            @pl.when(vb < _LAST)
            def _():
                lax.fori_loop(
                    0, 128,
                    lambda v, c: col(ins[b], v, lax.shift_right_logical(v, 1), c),
                    0, unroll=8,
                )

            @pl.when(vb == _LAST)
            def _():
                lax.fori_loop(
                    0, 64,
                    lambda v, c: col(tin, v, lax.shift_right_logical(v, 1), c),
                    0, unroll=8,
                )

            write_start(i, b)

    def outer(i, carry):
        step(i * 2, 0)
        step(i * 2 + 1, 1)
        return carry

    lax.fori_loop(0, (nit + 1) // 2, outer, 0)

    def drain(i, b):
        vb = wid + i * _NW

        @pl.when(vb <= _LAST)
        def _():
            write_wait(i, b)

    drain(nit - 2, (nit - 2) % 2)
    drain(nit - 1, (nit - 1) % 2)


def _b_body(ids_t, pairs, out, iv0, iv1, ho0, ho1, dzi,
            ga, gb, o0, o1, is0, is1, gs0, gs1, os0, os1):
    wid = lax.axis_index("s") * _NC + lax.axis_index("c")
    b0 = pl.multiple_of(wid * 128, 128)
    ivs = (iv0, iv1)
    hos = (ho0, ho1)
    gbufs = (ga, gb)
    obufs = (o0, o1)
    isems = (is0, is1)
    gsems = (gs0, gs1)
    osems = (os0, os1)

    def ids_read(l, b):
        return pltpu.make_async_copy(
            ids_t.at[l, pl.ds(b0, 128)], ivs[b], isems[b]
        )

    def gather_start(b):
        iv = ivs[b]
        ho = hos[b]
        for j in range(8):
            v = iv[pl.ds(j * 16, 16)]
            ho[pl.ds(j * 16, 16)] = lax.mul(lax.bitwise_and(v, 1), 64)
            px = lax.shift_right_logical(v, 1)
            pltpu.make_async_copy(pairs.at[px], gbufs[b][j], gsems[b]).start()

    def gather_wait(b):
        for j in range(8):
            pltpu.make_async_copy(
                pairs.at[dzi], gbufs[b][j], gsems[b]
            ).wait()

    def owrite(l, b):
        return pltpu.make_async_copy(
            obufs[b], out.at[l, :, pl.ds(b0, 128)], osems[b]
        )

    ids_read(0, 0).start()
    ids_read(0, 0).wait()
    gather_start(0)
    ids_read(1, 1).start()

    def step(l, b):
        nb = (b + 1) % 2

        @pl.when(l + 1 < _L)
        def _():
            ids_read(l + 1, nb).wait()
            gather_start(nb)

        @pl.when(l == 0)
        def _():
            gather_wait(0)
            gather_start(0)

        @pl.when(l + 2 < _L)
        def _():
            ids_read(l + 2, b).start()

        @pl.when(l >= 2)
        def _():
            owrite(l - 2, b).wait()

        gather_wait(b)

        o = obufs[b]
        hv = [hos[b][pl.ds(bg * 16, 16)] for bg in range(8)]
        rows16 = _iota16()

        def drow(d, carry):
            for bg in range(8):
                vals = plsc.load_gather(gbufs[b][bg], [rows16, hv[bg] + d])
                o[d, pl.ds(bg * 16, 16)] = vals
            return carry

        lax.fori_loop(0, 64, drow, 0, unroll=8)
        owrite(l, b).start()

    def outer(l2, carry):
        step(l2 * 2, 0)
        step(l2 * 2 + 1, 1)
        return carry

    lax.fori_loop(0, _L // 2, outer, 0)
    owrite(_L - 2, 0).wait()
    owrite(_L - 1, 1).wait()


def kernel(input_ids, table):
    mesh = plsc.VectorSubcoreMesh(
        core_axis_name="c", subcore_axis_name="s", num_cores=_NC, num_subcores=_NS
    )
    tab_t = table.T
    ids_t = input_ids.T

    pairs = pl.kernel(
        _a_body,
        out_type=jax.ShapeDtypeStruct((_V // 2, 128), jnp.float32),
        mesh=mesh,
        compiler_params=pltpu.CompilerParams(needs_layout_passes=False),
        scratch_types=(
            [pltpu.VMEM((_D, 128), jnp.float32)] * 2
            + [pltpu.VMEM((_D, _V - (_VBLK - 1) * 128), jnp.float32)]
            + [pltpu.VMEM((_D, 128), jnp.float32)] * 2
            + [pltpu.SemaphoreType.DMA] * 4
        ),
    )(tab_t)

    out3 = pl.kernel(
        _b_body,
        out_type=jax.ShapeDtypeStruct((_L, _D, _B), jnp.float32),
        mesh=mesh,
        compiler_params=pltpu.CompilerParams(needs_layout_passes=False),
        scratch_types=(
            [pltpu.VMEM((128,), jnp.int32)] * 4
            + [pltpu.VMEM((16,), jnp.int32)]
            + [tuple(pltpu.VMEM((16, 128), jnp.float32) for _ in range(8))] * 2
            + [pltpu.VMEM((_D, 128), jnp.float32)] * 2
            + [pltpu.SemaphoreType.DMA] * 6
        ),
    )(ids_t, pairs)

    return jnp.transpose(out3, (2, 0, 1))


# speedup vs baseline: 2.2355x; 2.2355x over previous
import jax
import jax.numpy as jnp
from jax import lax
from jax.experimental import pallas as pl
from jax.experimental.pallas import tpu as pltpu
from jax.experimental.pallas import tpu_sc as plsc

_NC = 2
_NS = 16
_NW = _NC * _NS
_CH = 128


_NBUF = 4
_LOOK = 2


def _body(idx_hbm, table_hbm, out_hbm, idx_v, *scratch):
    bufs = scratch[:_NBUF]
    gsems = scratch[_NBUF : 2 * _NBUF]
    wsems = scratch[2 * _NBUF :]
    nch = idx_v.shape[0]
    wid = lax.axis_index("s") * _NC + lax.axis_index("c")
    base = wid * (nch * _CH)

    pltpu.sync_copy(idx_hbm.at[wid], idx_v)

    def gather(j, b):
        return pltpu.make_async_copy(table_hbm.at[idx_v.at[j]], bufs[b], gsems[b])

    def writeback(j, b):
        return pltpu.make_async_copy(
            bufs[b], out_hbm.at[pl.ds(base + j * _CH, _CH)], wsems[b]
        )

    for c in range(_LOOK):
        gather(c, c).start()

    def step(j, b):
        jn = j + _LOOK
        bn = (b + _LOOK) % _NBUF

        @pl.when(jn < nch)
        def _():
            @pl.when(j >= _NBUF - _LOOK)
            def _():
                writeback(jn - _NBUF, bn).wait()

            gather(jn, bn).start()

        gather(j, b).wait()
        writeback(j, b).start()

    def outer(i, carry):
        for b in range(_NBUF):
            step(i * _NBUF + b, b)
        return carry

    lax.fori_loop(0, nch // _NBUF, outer, 0)

    for c in range(nch - _NBUF, nch):
        writeback(c, c % _NBUF).wait()


def kernel(input_ids, table):
    b, l = input_ids.shape
    dim = table.shape[1]
    total = b * l
    per_w = total // _NW
    nch = per_w // _CH
    idx = input_ids.reshape(_NW, nch, _CH)

    mesh = plsc.VectorSubcoreMesh(
        core_axis_name="c", subcore_axis_name="s", num_cores=_NC, num_subcores=_NS
    )
    out = pl.kernel(
        _body,
        out_type=jax.ShapeDtypeStruct((total, dim), jnp.float32),
        mesh=mesh,
        scratch_types=(
            [pltpu.VMEM((nch, _CH), jnp.int32)]
            + [pltpu.VMEM((_CH, dim), jnp.float32)] * _NBUF
            + [pltpu.SemaphoreType.DMA] * (2 * _NBUF)
        ),
        compiler_params=pltpu.CompilerParams(use_tc_tiling_on_sc=False),
    )(idx, table)
    return out.reshape(b, l, dim)
